# baseline (speedup 1.0000x reference)
import functools

import jax
import jax.numpy as jnp
from jax import lax
from jax.experimental import pallas as pl
from jax.experimental.pallas import tpu as pltpu
from jax.experimental.pallas import tpu_sc as plsc

B = 4096
L = 50
D = 64
NW = 32
NS = 16
BPW = B // NW
IPW = BPW * L
RPC = 1
CH = RPC * L
NCH = IPW // CH
NJ = D // 16
NBUF = 4
RUN = 8
DPAD = 56

_mesh = plsc.VectorSubcoreMesh(core_axis_name="c", subcore_axis_name="s")


@functools.partial(
    pl.kernel,
    mesh=_mesh,
    compiler_params=pltpu.CompilerParams(use_tc_tiling_on_sc=False),
    out_type=jax.ShapeDtypeStruct((B, D), jnp.float32),
    scratch_types=[
        pltpu.VMEM((BPW, L), jnp.int32),
        pltpu.VMEM((NCH * DPAD + 16,), jnp.int32),
        pltpu.VMEM((NBUF, CH, D), jnp.float32),
        pltpu.VMEM((BPW, D), jnp.float32),
        pltpu.VMEM_SHARED((NS * BPW, D), jnp.float32),
        pltpu.SemaphoreType.DMA,
        pltpu.SemaphoreType.DMA,
        pltpu.SemaphoreType.DMA,
        pltpu.SemaphoreType.DMA,
        pltpu.SemaphoreType.DMA,
    ],
)
def _mean_embed(table_hbm, x_hbm, out_hbm,
                x2_v, d_v, rows_v, stage_v, acc_sh,
                gs0, gs1, gs2, gs3, ssem):
    cid = lax.axis_index("c")
    sid = lax.axis_index("s")
    wid = sid * 2 + cid
    pltpu.sync_copy(x_hbm.at[pl.ds(wid * BPW, BPW)], x2_v)
    gsems = (gs0, gs1, gs2, gs3)

    base = (sid * BPW).astype(jnp.int32)
    zero = jnp.zeros((16,), jnp.float32)
    onei = jnp.zeros((16,), jnp.int32)

    def dst_block(c, carry):
        val = onei + (base + c)
        for j in range(4):
            d_v[pl.ds(c * DPAD + j * 16, 16)] = val
        return carry

    lax.fori_loop(0, NCH, dst_block, 0)

    def zero_block(g, carry):
        for rr in range(RUN):
            for j in range(NJ):
                stage_v[g * RUN + rr, pl.ds(j * 16, 16)] = zero
        return carry

    lax.fori_loop(0, BPW // RUN, zero_block, 0)
    pltpu.sync_copy(stage_v, acc_sh.at[pl.ds(sid * BPW, BPW)])

    def gather(c, b):
        return pltpu.make_async_copy(
            table_hbm.at[x2_v.at[c]], rows_v.at[b], gsems[b])

    for b in range(NBUF):
        gather(b, b).start()

    def step(i, carry):
        for b in range(NBUF):
            c = i * NBUF + b
            gather(c, b).wait()
            pltpu.async_copy(
                rows_v.at[b], acc_sh.at[d_v.at[pl.ds(c * DPAD, CH)]], ssem,
                add=True).wait()

            @pl.when(c + NBUF < NCH)
            def _():
                gather(c + NBUF, b).start()

        return carry

    lax.fori_loop(0, NCH // NBUF, step, 0)
    pltpu.sync_copy(acc_sh.at[pl.ds(sid * BPW, BPW)], stage_v)

    def scale_block(g, carry):
        for rr in range(RUN):
            r = g * RUN + rr
            for j in range(NJ):
                stage_v[r, pl.ds(j * 16, 16)] = (
                    stage_v[r, pl.ds(j * 16, 16)] * (1.0 / L))
        return carry

    lax.fori_loop(0, BPW // RUN, scale_block, 0)
    pltpu.sync_copy(stage_v, out_hbm.at[pl.ds(wid * BPW, BPW)])


def kernel(vectors, x):
    return _mean_embed(vectors, x.astype(jnp.int32))
